# baseline (speedup 1.0000x reference)
import functools

import jax
import jax.numpy as jnp
from jax import lax
from jax.experimental import pallas as pl
from jax.experimental.pallas import tpu as pltpu
from jax.experimental.pallas import tpu_sc as plsc

N_ROWS = 16384
D = 4096
NUM_WORKERS = 32
ROWS_PER_W = N_ROWS // NUM_WORKERS
R = 2
CHUNK = R * D
NCHUNK = ROWS_PER_W // R
NB = 4
LANES = 16


def _make_kernel():
    mesh = plsc.VectorSubcoreMesh(core_axis_name="c", subcore_axis_name="s")

    @functools.partial(
        pl.kernel,
        out_type=jax.ShapeDtypeStruct((N_ROWS * D,), jnp.float32),
        mesh=mesh,
        scratch_types=[
            pltpu.VMEM((D,), jnp.int32),
            pltpu.VMEM((NB, CHUNK), jnp.float32),
            pltpu.VMEM((NB, CHUNK), jnp.float32),
            pltpu.SemaphoreType.DMA,
            pltpu.SemaphoreType.DMA,
            pltpu.SemaphoreType.DMA,
            pltpu.SemaphoreType.DMA,
            pltpu.SemaphoreType.DMA,
            pltpu.SemaphoreType.DMA,
            pltpu.SemaphoreType.DMA,
            pltpu.SemaphoreType.DMA,
        ],
        compiler_params=pltpu.CompilerParams(
            use_tc_tiling_on_sc=False, needs_layout_passes=False
        ),
    )
    def run(z_hbm, p_hbm, out_hbm, p_v, in_v, out_v,
            si0, si1, si2, si3, so0, so1, so2, so3):
        wid = lax.axis_index("s") * 2 + lax.axis_index("c")
        base = wid * ROWS_PER_W * D
        pltpu.sync_copy(p_hbm, p_v)

        isems = (si0, si1, si2, si3)
        osems = (so0, so1, so2, so3)

        def start_in(c, b):
            pltpu.async_copy(z_hbm.at[pl.ds(base + c * CHUNK, CHUNK)],
                             in_v.at[b], isems[b])

        def wait_in(c, b):
            pltpu.make_async_copy(z_hbm.at[pl.ds(base + c * CHUNK, CHUNK)],
                                  in_v.at[b], isems[b]).wait()

        def start_out(c, b):
            pltpu.async_copy(out_v.at[b],
                             out_hbm.at[pl.ds(base + c * CHUNK, CHUNK)],
                             osems[b])

        def wait_out(c, b):
            pltpu.make_async_copy(out_v.at[b],
                                  out_hbm.at[pl.ds(base + c * CHUNK, CHUNK)],
                                  osems[b]).wait()

        def gather(b):
            ib = in_v.at[b]
            ob = out_v.at[b]

            @plsc.parallel_loop(0, D // LANES, 1, unroll=4)
            def jloop(j):
                jb = j * LANES
                cols = p_v[pl.ds(jb, LANES)]
                for r in range(R):
                    vals = plsc.load_gather(ib.at[pl.ds(r * D, D)], [cols])
                    ob[pl.ds(r * D + jb, LANES)] = vals

        for b in range(NB):
            start_in(b, b)

        for b in range(NB):
            wait_in(b, b)
            gather(b)
            start_in(b + NB, b)
            start_out(b, b)

        def body(g, carry):
            for b in range(NB):
                c = g * NB + b
                wait_in(c, b)
                wait_out(c - NB, b)
                gather(b)
                start_in(c + NB, b)
                start_out(c, b)
            return carry

        lax.fori_loop(1, NCHUNK // NB - 1, body, 0)

        for b in range(NB):
            c = NCHUNK - NB + b
            wait_in(c, b)
            wait_out(c - NB, b)
            gather(b)
            start_out(c, b)
        for b in range(NB):
            wait_out(NCHUNK - NB + b, b)

    return run


_sc_permute = _make_kernel()


def kernel(z, P):
    out = _sc_permute(z.reshape(-1), P.astype(jnp.int32))
    return out.reshape(N_ROWS, D)


# speedup vs baseline: 1.8457x; 1.8457x over previous
import functools

import jax
import jax.numpy as jnp
from jax import lax
from jax.experimental import pallas as pl
from jax.experimental.pallas import tpu as pltpu
from jax.experimental.pallas import tpu_sc as plsc

N_ROWS = 16384
D = 4096
NUM_WORKERS = 32
ROWS_PER_W = N_ROWS // NUM_WORKERS
R = 8
NCHUNK = ROWS_PER_W // R
LANES = 16


def _make_kernel():
    mesh = plsc.VectorSubcoreMesh(core_axis_name="c", subcore_axis_name="s")

    @functools.partial(
        pl.kernel,
        out_type=jax.ShapeDtypeStruct((N_ROWS, D), jnp.float32),
        mesh=mesh,
        scratch_types=[
            pltpu.VMEM((D,), jnp.int32),
            pltpu.VMEM((R, D), jnp.float32),
            pltpu.VMEM((R, D), jnp.float32),
        ],
        compiler_params=pltpu.CompilerParams(
            use_tc_tiling_on_sc=True, needs_layout_passes=False
        ),
    )
    def run(z_hbm, p_hbm, out_hbm, p_v, in_v, out_v):
        wid = lax.axis_index("s") * 2 + lax.axis_index("c")
        base = wid * ROWS_PER_W
        pltpu.sync_copy(p_hbm, p_v)

        def chunk(c, carry):
            row0 = base + c * R
            pltpu.sync_copy(z_hbm.at[pl.ds(row0, R)], in_v)

            @plsc.parallel_loop(0, D // LANES, 1, unroll=4)
            def jloop(j):
                jb = j * LANES
                cols = p_v[pl.ds(jb, LANES)]
                for r in range(R):
                    rows = jnp.full((LANES,), r, jnp.int32)
                    vals = plsc.load_gather(in_v, [rows, cols])
                    out_v[r, pl.ds(jb, LANES)] = vals

            pltpu.sync_copy(out_v, out_hbm.at[pl.ds(row0, R)])
            return carry

        lax.fori_loop(0, NCHUNK, chunk, 0)

    return run


_sc_permute = _make_kernel()


def kernel(z, P):
    return _sc_permute(z, P.astype(jnp.int32))


# speedup vs baseline: 3.1275x; 1.6945x over previous
import functools

import jax
import jax.numpy as jnp
from jax import lax
from jax.experimental import pallas as pl
from jax.experimental.pallas import tpu as pltpu
from jax.experimental.pallas import tpu_sc as plsc

N_ROWS = 16384
D = 4096
HALF = D // 2
NUM_WORKERS = 32
ROWS_PER_W = N_ROWS // NUM_WORKERS
R = 8
NCHUNK = ROWS_PER_W // R
LANES = 16


def _make_kernel():
    mesh = plsc.VectorSubcoreMesh(core_axis_name="c", subcore_axis_name="s")

    @functools.partial(
        pl.kernel,
        out_type=jax.ShapeDtypeStruct((N_ROWS, D), jnp.float32),
        mesh=mesh,
        scratch_types=[
            pltpu.VMEM((D,), jnp.int32),
            pltpu.VMEM((2, R, D), jnp.float32),
            pltpu.VMEM((2, R, HALF), jnp.float32),
            pltpu.SemaphoreType.DMA,
            pltpu.SemaphoreType.DMA,
            pltpu.SemaphoreType.DMA,
            pltpu.SemaphoreType.DMA,
        ],
        compiler_params=pltpu.CompilerParams(
            use_tc_tiling_on_sc=True, needs_layout_passes=False
        ),
    )
    def run(z_hbm, p_hbm, out_hbm, p_v, in_v, out_v, si0, si1, so0, so1):
        wid = lax.axis_index("s") * 2 + lax.axis_index("c")
        base = wid * ROWS_PER_W
        pltpu.sync_copy(p_hbm, p_v)
        isems = (si0, si1)
        osems = (so0, so1)

        def start_in(c, b):
            pltpu.async_copy(z_hbm.at[pl.ds(base + c * R, R)],
                             in_v.at[b], isems[b])

        def wait_in(c, b):
            pltpu.make_async_copy(z_hbm.at[pl.ds(base + c * R, R)],
                                  in_v.at[b], isems[b]).wait()

        def start_out(c, h):
            pltpu.async_copy(
                out_v.at[h],
                out_hbm.at[pl.ds(base + c * R, R), pl.ds(h * HALF, HALF)],
                osems[h])

        def wait_out(c, h):
            pltpu.make_async_copy(
                out_v.at[h],
                out_hbm.at[pl.ds(base + c * R, R), pl.ds(h * HALF, HALF)],
                osems[h]).wait()

        def gather_half(b, h):
            ib = in_v.at[b]
            ob = out_v.at[h]

            @plsc.parallel_loop(0, HALF // LANES, 1, unroll=4)
            def jloop(j):
                jb = j * LANES
                cols = p_v[pl.ds(h * HALF + jb, LANES)]
                for r in range(R):
                    rows = jnp.full((LANES,), r, jnp.int32)
                    vals = plsc.load_gather(ib, [rows, cols])
                    ob[r, pl.ds(jb, LANES)] = vals

        start_in(0, 0)
        start_in(1, 1)
        wait_in(0, 0)
        for h in range(2):
            gather_half(0, h)
            start_out(0, h)
        start_in(2, 0)

        def pair(g, carry):
            for b in range(2):
                c = g * 2 + b
                wait_in(c, b)
                for h in range(2):
                    wait_out(c - 1, h)
                    gather_half(b, h)
                    start_out(c, h)
                start_in(c + 2, b)
            return carry

        c = 1
        wait_in(c, 1)
        for h in range(2):
            wait_out(c - 1, h)
            gather_half(1, h)
            start_out(c, h)
        start_in(c + 2, 1)

        lax.fori_loop(1, NCHUNK // 2 - 1, pair, 0)

        for b in range(2):
            c = NCHUNK - 2 + b
            wait_in(c, b)
            for h in range(2):
                wait_out(c - 1, h)
                gather_half(b, h)
                start_out(c, h)
        for h in range(2):
            wait_out(NCHUNK - 1, h)

    return run


_sc_permute = _make_kernel()


def kernel(z, P):
    return _sc_permute(z, P.astype(jnp.int32))
